# Initial kernel scaffold; baseline (speedup 1.0000x reference)
import jax
import jax.numpy as jnp
from jax.experimental import pallas as pl


def kernel(x, edge_index, Wl0, bl0, Wr0, Wl1, bl1, Wr1, Wl2, bl2, Wr2, Wl3, bl3, Wr3):
    raise NotImplementedError("write your pallas kernel here")


# speedup vs baseline: 8.3483x; 8.3483x over previous
import functools

import jax
import jax.numpy as jnp
from jax import lax
from jax.experimental import pallas as pl
from jax.experimental.pallas import tpu as pltpu
from jax.experimental.pallas import tpu_sc as plsc

N = 10000
NP = 10240
E = 320000
DUMMY = N
LDIMS = [(128, 85), (85, 56), (56, 28), (28, 1)]
WIN = [128, 96, 64, 32]
WOUT = [96, 64, 32, 16]
NW = 32
CHUNK = 128
NCH = 79
EPT = NCH * CHUNK
EP = NW * EPT
RPT = NP // 16
TCB = 1024


def _make_sc_seg_sum(wp):
  mesh = plsc.VectorSubcoreMesh(core_axis_name="c", subcore_axis_name="s")

  @functools.partial(
      pl.kernel,
      mesh=mesh,
      compiler_params=pltpu.CompilerParams(use_tc_tiling_on_sc=False),
      out_type=(
          jax.ShapeDtypeStruct((NP, wp), jnp.float32),
          jax.ShapeDtypeStruct((NP, wp), jnp.float32),
      ),
      scratch_types=[
          pltpu.VMEM((NCH, CHUNK), jnp.int32),
          pltpu.VMEM((NCH, CHUNK), jnp.int32),
          pltpu.VMEM((CHUNK, wp), jnp.float32),
          pltpu.VMEM_SHARED((NP, wp), jnp.float32),
          pltpu.SemaphoreType.DMA,
      ],
  )
  def seg_sum(p_hbm, src_hbm, dst_hbm, zero_hbm, out_a, out_b,
              src_v, dst_v, rows_v, acc, sem):
    c = lax.axis_index("c")
    s = lax.axis_index("s")
    wid = c * 16 + s
    r0 = s * RPT
    pltpu.sync_copy(zero_hbm.at[pl.ds(r0, RPT)], acc.at[pl.ds(r0, RPT)])
    pltpu.sync_copy(src_hbm.at[wid], src_v)
    pltpu.sync_copy(dst_hbm.at[wid], dst_v)
    plsc.subcore_barrier()

    def body(j, carry):
      pltpu.async_copy(p_hbm.at[src_v.at[j]], rows_v, sem).wait()
      pltpu.sync_copy(rows_v, acc.at[dst_v.at[j]], add=True)
      return carry

    lax.fori_loop(0, NCH, body, 0)
    plsc.subcore_barrier()

    @pl.when(c == 0)
    def _():
      pltpu.sync_copy(acc.at[pl.ds(r0, RPT)], out_a.at[pl.ds(r0, RPT)])

    @pl.when(c == 1)
    def _():
      pltpu.sync_copy(acc.at[pl.ds(r0, RPT)], out_b.at[pl.ds(r0, RPT)])

  return seg_sum


def _mm_body(x_ref, wl_ref, wr_ref, cv_ref, bv_ref, p_ref, q_ref):
  x = x_ref[...]
  p_ref[...] = jnp.dot(x, wl_ref[...],
                       preferred_element_type=jnp.float32) + cv_ref[...]
  q_ref[...] = jnp.dot(x, wr_ref[...],
                       preferred_element_type=jnp.float32) + bv_ref[...]


def _comb_body(dcol, sa_ref, sb_ref, qp_ref, wl_ref, wr_ref, cv_ref, bv_ref,
               p_ref, q_ref):
  ssum = sa_ref[...] + sb_ref[...]
  cnt = ssum[:, dcol:dcol + 1]
  inv = 1.0 / jnp.maximum(cnt, 1.0)
  h = ssum * inv + qp_ref[...]
  p_ref[...] = jnp.dot(h, wl_ref[...],
                       preferred_element_type=jnp.float32) + cv_ref[...]
  q_ref[...] = jnp.dot(h, wr_ref[...],
                       preferred_element_type=jnp.float32) + bv_ref[...]


def _final_body(sa_ref, sb_ref, qp_ref, mask_ref, out_ref):
  ssum = sa_ref[...] + sb_ref[...]
  cnt = ssum[:, 1:2]
  inv = 1.0 / jnp.maximum(cnt, 1.0)
  out_ref[...] = (ssum * inv + qp_ref[...]) * mask_ref[...]


def _row_spec(w):
  return pl.BlockSpec((TCB, w), lambda i: (i, 0))


def _full_spec(r, cm):
  return pl.BlockSpec((r, cm), lambda i: (0, 0))


def _transform_call(x, wl, wr, cv, bv):
  win, wout = wl.shape
  return pl.pallas_call(
      _mm_body,
      grid=(NP // TCB,),
      in_specs=[
          _row_spec(win),
          _full_spec(win, wout),
          _full_spec(win, wout),
          _full_spec(1, wout),
          _full_spec(1, wout),
      ],
      out_specs=[_row_spec(wout), _row_spec(wout)],
      out_shape=[jax.ShapeDtypeStruct((NP, wout), jnp.float32)] * 2,
  )(x, wl, wr, cv, bv)


def _combine_call(sa, sb, qp, wl, wr, cv, bv, dcol):
  win, wout = wl.shape
  return pl.pallas_call(
      functools.partial(_comb_body, dcol),
      grid=(NP // TCB,),
      in_specs=[
          _row_spec(win),
          _row_spec(win),
          _row_spec(win),
          _full_spec(win, wout),
          _full_spec(win, wout),
          _full_spec(1, wout),
          _full_spec(1, wout),
      ],
      out_specs=[_row_spec(wout), _row_spec(wout)],
      out_shape=[jax.ShapeDtypeStruct((NP, wout), jnp.float32)] * 2,
  )(sa, sb, qp, wl, wr, cv, bv)


def _final_call(sa, sb, qp, mask):
  w = sa.shape[1]
  return pl.pallas_call(
      _final_body,
      grid=(NP // TCB,),
      in_specs=[
          _row_spec(w),
          _row_spec(w),
          _row_spec(w),
          _full_spec(1, w),
      ],
      out_specs=_row_spec(w),
      out_shape=jax.ShapeDtypeStruct((NP, w), jnp.float32),
  )(sa, sb, qp, mask)


def kernel(x, edge_index, Wl0, bl0, Wr0, Wl1, bl1, Wr1, Wl2, bl2, Wr2,
           Wl3, bl3, Wr3):
  f32 = jnp.float32
  ei = edge_index.astype(jnp.int32)
  pad_e = EP - E
  pad_idx = jnp.full((pad_e,), DUMMY, jnp.int32)
  src = jnp.concatenate([ei[0], pad_idx]).reshape(NW, NCH, CHUNK)
  dst = jnp.concatenate([ei[1], pad_idx]).reshape(NW, NCH, CHUNK)
  xp = jnp.zeros((NP, 128), f32).at[:N].set(x)

  wls = [Wl0, Wl1, Wl2, Wl3]
  bls = [bl0, bl1, bl2, bl3]
  wrs = [Wr0, Wr1, Wr2, Wr3]
  wl_p, wr_p, cv_p, bv_p = [], [], [], []
  for l, (din, dout) in enumerate(LDIMS):
    win, wout = WIN[l], WOUT[l]
    wl_p.append(jnp.zeros((win, wout), f32).at[:din, :dout].set(wls[l].T))
    wr_p.append(jnp.zeros((win, wout), f32).at[:din, :dout].set(wrs[l].T))
    cv_p.append(jnp.zeros((1, wout), f32).at[0, dout].set(1.0))
    bv_p.append(jnp.zeros((1, wout), f32).at[0, :dout].set(bls[l]))

  sc_calls = [_make_sc_seg_sum(w) for w in WOUT]

  p, q = _transform_call(xp, wl_p[0], wr_p[0], cv_p[0], bv_p[0])
  out = None
  for l in range(4):
    zeros_l = jnp.zeros((NP, WOUT[l]), f32)
    sa, sb = sc_calls[l](p, src, dst, zeros_l)
    if l < 3:
      p, q = _combine_call(sa, sb, q, wl_p[l + 1], wr_p[l + 1],
                           cv_p[l + 1], bv_p[l + 1], LDIMS[l + 1][0])
    else:
      mask = jnp.zeros((1, WOUT[3]), f32).at[0, 0].set(1.0)
      out = _final_call(sa, sb, q, mask)
  return out[:N, 0:1]
